# Initial kernel scaffold; baseline (speedup 1.0000x reference)
import jax
import jax.numpy as jnp
from jax.experimental import pallas as pl


def kernel(objects, vertices, rp_param, cond_param, sc_param, xpd_param):
    raise NotImplementedError("write your pallas kernel here")


# speedup vs baseline: 302.6125x; 302.6125x over previous
import functools

import numpy as np
import jax
import jax.numpy as jnp
from jax import lax
from jax.experimental import pallas as pl
from jax.experimental.pallas import tpu as pltpu
from jax.experimental.pallas import tpu_sc as plsc

_NUM_MATERIALS = 1000
_TBL = 1024
_FREQUENCY = 3.5e9
_EPS0 = 8.8541878128e-12
_NEG_INV_EPS_OMEGA = np.float32(
    -1.0 / (_EPS0 * np.float32(2.0 * np.pi * _FREQUENCY))
)

_L = 16
_NC, _NS = 2, 16
_NW = _NC * _NS
_N = 4 * 1048576
_PER_W = _N // _NW
_C = 8192
_NCHUNK = _PER_W // _C


def _body(obj_hbm, rp_hbm, cond_hbm, sch_hbm, xpd_hbm,
          ore_hbm, oim_hbm, osc_hbm, oxp_hbm,
          tbl_re, tbl_im, tbl_sc, tbl_xp,
          idx_v, ore_v, oim_v, osc_v, oxp_v):
    cid = lax.axis_index("c")
    sid = lax.axis_index("s")
    wid = sid * _NC + cid
    base = wid * _PER_W

    pltpu.sync_copy(rp_hbm, tbl_re)
    pltpu.sync_copy(cond_hbm, tbl_im)
    pltpu.sync_copy(sch_hbm, tbl_sc)
    pltpu.sync_copy(xpd_hbm, tbl_xp)

    lanes = lax.iota(jnp.int32, _L)

    @plsc.parallel_loop(0, _TBL, _L)
    def _activate(r):
        rows = lanes + r
        sent = rows == _NUM_MATERIALS
        p = tbl_re[pl.ds(r, _L)]
        tbl_re[pl.ds(r, _L)] = jnp.where(sent, 1000.0, 2.0 + jnp.exp(p))
        q = tbl_im[pl.ds(r, _L)]
        tbl_im[pl.ds(r, _L)] = jnp.where(
            sent, 0.0, jnp.exp(q) * _NEG_INV_EPS_OMEGA)
        s = tbl_sc[pl.ds(r, _L)]
        tbl_sc[pl.ds(r, _L)] = jnp.where(
            sent, 1000.0, 1.0 / (1.0 + jnp.exp(-s)))
        x = tbl_xp[pl.ds(r, _L)]
        tbl_xp[pl.ds(r, _L)] = jnp.where(
            sent, 1000.0, 1.0 / (1.0 + jnp.exp(-x)))

    for k in range(_NCHUNK):
        off = base + k * _C
        pltpu.sync_copy(obj_hbm.at[pl.ds(off, _C)], idx_v)

        @plsc.parallel_loop(0, _C, _L, unroll=8)
        def _gather(g):
            iv = idx_v[pl.ds(g, _L)]
            iv = jnp.where(iv < 0, _NUM_MATERIALS, iv)
            ore_v[pl.ds(g, _L)] = plsc.load_gather(tbl_re, [iv])
            oim_v[pl.ds(g, _L)] = plsc.load_gather(tbl_im, [iv])
            osc_v[pl.ds(g, _L)] = plsc.load_gather(tbl_sc, [iv])
            oxp_v[pl.ds(g, _L)] = plsc.load_gather(tbl_xp, [iv])

        pltpu.sync_copy(ore_v, ore_hbm.at[pl.ds(off, _C)])
        pltpu.sync_copy(oim_v, oim_hbm.at[pl.ds(off, _C)])
        pltpu.sync_copy(osc_v, osc_hbm.at[pl.ds(off, _C)])
        pltpu.sync_copy(oxp_v, oxp_hbm.at[pl.ds(off, _C)])


def kernel(objects, vertices, rp_param, cond_param, sc_param, xpd_param):
    del vertices
    shape = objects.shape
    obj_flat = objects.reshape(_N)
    pad = _TBL - _NUM_MATERIALS
    rp_p = jnp.pad(rp_param, (0, pad))
    cond_p = jnp.pad(cond_param, (0, pad))
    sc_p = jnp.pad(sc_param, (0, pad))
    xpd_p = jnp.pad(xpd_param, (0, pad))

    mesh = plsc.VectorSubcoreMesh(
        core_axis_name="c", subcore_axis_name="s",
        num_cores=_NC, num_subcores=_NS)
    f32 = jnp.float32
    out_type = (
        jax.ShapeDtypeStruct((_N,), f32),
        jax.ShapeDtypeStruct((_N,), f32),
        jax.ShapeDtypeStruct((_N,), f32),
        jax.ShapeDtypeStruct((_N,), f32),
    )
    scratch = [
        pltpu.VMEM((_TBL,), f32),
        pltpu.VMEM((_TBL,), f32),
        pltpu.VMEM((_TBL,), f32),
        pltpu.VMEM((_TBL,), f32),
        pltpu.VMEM((_C,), jnp.int32),
        pltpu.VMEM((_C,), f32),
        pltpu.VMEM((_C,), f32),
        pltpu.VMEM((_C,), f32),
        pltpu.VMEM((_C,), f32),
    ]
    run = pl.kernel(_body, out_type=out_type, mesh=mesh,
                    scratch_types=scratch,
                    compiler_params=pltpu.CompilerParams(
                        needs_layout_passes=False))
    re, im, scg, xpg = run(obj_flat, rp_p, cond_p, sc_p, xpd_p)
    crp = lax.complex(re, im).reshape(shape)
    return crp, scg.reshape(shape), xpg.reshape(shape)


# speedup vs baseline: 428.4028x; 1.4157x over previous
import numpy as np
import jax
import jax.numpy as jnp
from jax import lax
from jax.experimental import pallas as pl
from jax.experimental.pallas import tpu as pltpu
from jax.experimental.pallas import tpu_sc as plsc

_NUM_MATERIALS = 1000
_TBL = 1024
_FREQUENCY = 3.5e9
_EPS0 = 8.8541878128e-12
_NEG_INV_EPS_OMEGA = np.float32(
    -1.0 / (_EPS0 * np.float32(2.0 * np.pi * _FREQUENCY))
)

_L = 16
_NC, _NS = 2, 16
_NW = _NC * _NS
_N = 4 * 1048576
_PER_W = _N // _NW
_C = 8192
_NCHUNK = _PER_W // _C


def _body(obj_hbm, rp_hbm, cond_hbm, sch_hbm, xpd_hbm,
          ore_hbm, oim_hbm, osc_hbm, oxp_hbm,
          tbl_re, tbl_im, tbl_sc, tbl_xp,
          idx_v0, idx_v1, ore_v0, oim_v0, osc_v0, oxp_v0,
          ore_v1, oim_v1, osc_v1, oxp_v1,
          ore_v2, oim_v2, osc_v2, oxp_v2,
          sem_in0, sem_in1, sem_out0, sem_out1, sem_out2):
    cid = lax.axis_index("c")
    sid = lax.axis_index("s")
    wid = sid * _NC + cid
    base = wid * _PER_W
    idx_v = (idx_v0, idx_v1)
    outs_v = ((ore_v0, oim_v0, osc_v0, oxp_v0),
              (ore_v1, oim_v1, osc_v1, oxp_v1),
              (ore_v2, oim_v2, osc_v2, oxp_v2))
    sem_in = (sem_in0, sem_in1)
    sem_out = (sem_out0, sem_out1, sem_out2)
    outs_hbm = (ore_hbm, oim_hbm, osc_hbm, oxp_hbm)

    pltpu.sync_copy(rp_hbm, tbl_re.at[pl.ds(0, _NUM_MATERIALS)])
    pltpu.sync_copy(cond_hbm, tbl_im.at[pl.ds(0, _NUM_MATERIALS)])
    pltpu.sync_copy(sch_hbm, tbl_sc.at[pl.ds(0, _NUM_MATERIALS)])
    pltpu.sync_copy(xpd_hbm, tbl_xp.at[pl.ds(0, _NUM_MATERIALS)])

    lanes = lax.iota(jnp.int32, _L)

    @plsc.parallel_loop(0, _TBL, _L)
    def _activate(r):
        rows = lanes + r
        sent = rows == _NUM_MATERIALS
        p = tbl_re[pl.ds(r, _L)]
        tbl_re[pl.ds(r, _L)] = jnp.where(sent, 1000.0, 2.0 + jnp.exp(p))
        q = tbl_im[pl.ds(r, _L)]
        tbl_im[pl.ds(r, _L)] = jnp.where(
            sent, 0.0, jnp.exp(q) * _NEG_INV_EPS_OMEGA)
        s = tbl_sc[pl.ds(r, _L)]
        tbl_sc[pl.ds(r, _L)] = jnp.where(
            sent, 1000.0, 1.0 / (1.0 + jnp.exp(-s)))
        x = tbl_xp[pl.ds(r, _L)]
        tbl_xp[pl.ds(r, _L)] = jnp.where(
            sent, 1000.0, 1.0 / (1.0 + jnp.exp(-x)))

    pltpu.async_copy(obj_hbm.at[pl.ds(base, _C)], idx_v[0], sem_in[0])
    out_dmas = [None, None, None]
    for k in range(_NCHUNK):
        b = k % 2
        bo = k % 3
        off = base + k * _C
        if k + 1 < _NCHUNK:
            pltpu.async_copy(obj_hbm.at[pl.ds(off + _C, _C)],
                             idx_v[1 - b], sem_in[1 - b])
        if out_dmas[bo] is not None:
            for d in out_dmas[bo]:
                d.wait()
        pltpu.make_async_copy(obj_hbm.at[pl.ds(off, _C)], idx_v[b],
                              sem_in[b]).wait()
        ib = idx_v[b]
        rb, mb, sb, xb = outs_v[bo]

        @plsc.parallel_loop(0, _C, _L, unroll=8)
        def _gather(g):
            iv = ib[pl.ds(g, _L)]
            iv = jnp.where(iv < 0, _NUM_MATERIALS, iv)
            rb[pl.ds(g, _L)] = plsc.load_gather(tbl_re, [iv])
            mb[pl.ds(g, _L)] = plsc.load_gather(tbl_im, [iv])
            sb[pl.ds(g, _L)] = plsc.load_gather(tbl_sc, [iv])
            xb[pl.ds(g, _L)] = plsc.load_gather(tbl_xp, [iv])

        out_dmas[bo] = [
            pltpu.async_copy(v, h.at[pl.ds(off, _C)], sem_out[bo])
            for v, h in zip(outs_v[bo], outs_hbm)
        ]
    for dmas in out_dmas:
        if dmas is not None:
            for d in dmas:
                d.wait()


def _untile(x, shape):
    return x.reshape(shape[1] // 128, shape[0], 128).transpose(1, 0, 2) \
            .reshape(shape)


def kernel(objects, vertices, rp_param, cond_param, sc_param, xpd_param):
    del vertices
    shape = objects.shape
    obj_flat = objects.reshape(shape[0], shape[1] // 128, 128) \
                      .transpose(1, 0, 2).reshape(_N)
    mesh = plsc.VectorSubcoreMesh(
        core_axis_name="c", subcore_axis_name="s",
        num_cores=_NC, num_subcores=_NS)
    f32 = jnp.float32
    out_type = (
        jax.ShapeDtypeStruct((_N,), f32),
        jax.ShapeDtypeStruct((_N,), f32),
        jax.ShapeDtypeStruct((_N,), f32),
        jax.ShapeDtypeStruct((_N,), f32),
    )
    scratch = (
        [pltpu.VMEM((_TBL,), f32)] * 4
        + [pltpu.VMEM((_C,), jnp.int32)] * 2
        + [pltpu.VMEM((_C,), f32)] * 12
        + [pltpu.SemaphoreType.DMA] * 5
    )
    run = pl.kernel(_body, out_type=out_type, mesh=mesh,
                    scratch_types=scratch,
                    compiler_params=pltpu.CompilerParams(
                        needs_layout_passes=False))
    re, im, scg, xpg = run(obj_flat, rp_param, cond_param,
                           sc_param, xpd_param)
    crp = _untile(lax.complex(re, im), shape)
    return crp, _untile(scg, shape), _untile(xpg, shape)
